# baseline (speedup 1.0000x reference)
import functools

import jax
import jax.numpy as jnp
from jax import lax
from jax.experimental import pallas as pl
from jax.experimental.pallas import tpu as pltpu
from jax.experimental.pallas import tpu_sc as plsc

NUM_EXPERTS = 8
TOP_K = 2
D_MODEL = 1024
D_HID = 2048
AUX_COEFF = 0.01

T_TOKENS = 2048
N_PAIRS = T_TOKENS * TOP_K
BM = 256
NB = (N_PAIRS + NUM_EXPERTS * BM) // BM
P_ROWS = NB * BM
BH = 512
NHB = D_HID // BH

NW = 32
TPW = T_TOKENS // NW


def _cumsum0(a):
    n = a.shape[0]
    sh = 1
    while sh < n:
        a = a + jnp.concatenate(
            [jnp.zeros((sh, a.shape[1]), a.dtype), a[:-sh]], axis=0)
        sh *= 2
    return a


def _router_body(x_ref, gwt_ref, gb_ref, slots_ref, gates_ref, plen_ref,
                 aux_ref):
    T, E = T_TOKENS, NUM_EXPERTS
    logits = jnp.dot(x_ref[...], gwt_ref[...],
                     preferred_element_type=jnp.float32) + gb_ref[...]
    ids = jax.lax.broadcasted_iota(jnp.int32, (T, E), 1)
    m1 = jnp.max(logits, axis=1, keepdims=True)
    i1 = jnp.min(jnp.where(logits == m1, ids, E), axis=1, keepdims=True)
    neg = jnp.float32(-jnp.inf)
    logits_m = jnp.where(ids == i1, neg, logits)
    m2 = jnp.max(logits_m, axis=1, keepdims=True)
    i2 = jnp.min(jnp.where(logits_m == m2, ids, E), axis=1, keepdims=True)
    e21 = jnp.exp(m2 - m1)
    g1 = 1.0 / (1.0 + e21)
    g2 = e21 / (1.0 + e21)
    gates_ref[...] = jnp.concatenate([g1, g2], axis=1)
    lse = m1 + jnp.log(jnp.sum(jnp.exp(logits - m1), axis=1, keepdims=True))
    aux = (AUX_COEFF / E) * (-jnp.log(jnp.float32(E))
                             - jnp.mean(logits) + jnp.mean(lse))
    aux_ref[...] = jnp.reshape(aux, (1, 1))
    oh1 = (ids == i1).astype(jnp.int32)
    oh2 = (ids == i2).astype(jnp.int32)
    csum = _cumsum0(jnp.concatenate([oh1, oh2], axis=0))
    rank1 = jnp.sum(csum[:T] * oh1, axis=1, keepdims=True) - 1
    rank2 = jnp.sum(csum[T:] * oh2, axis=1, keepdims=True) - 1
    counts = csum[2 * T - 1:2 * T, :]
    plen = ((counts + (BM - 1)) // BM) * BM
    plen_ref[...] = plen
    ec = jax.lax.broadcasted_iota(jnp.int32, (E, E), 1)
    er = jax.lax.broadcasted_iota(jnp.int32, (E, E), 0)
    off = jnp.sum(jnp.where(ec < er, jnp.broadcast_to(plen, (E, E)), 0),
                  axis=1).reshape(1, E)
    off_b = jnp.broadcast_to(off, (T, E))
    slot1 = jnp.sum(oh1 * off_b, axis=1, keepdims=True) + rank1
    slot2 = jnp.sum(oh2 * off_b, axis=1, keepdims=True) + rank2
    slots_ref[...] = jnp.concatenate([slot1, slot2], axis=1)


def _router(x2d, gate_w, gate_b, interpret=False):
    T, E = T_TOKENS, NUM_EXPERTS
    slots, gates, plen, aux = pl.pallas_call(
        _router_body,
        out_shape=(jax.ShapeDtypeStruct((T, TOP_K), jnp.int32),
                   jax.ShapeDtypeStruct((T, TOP_K), jnp.float32),
                   jax.ShapeDtypeStruct((1, E), jnp.int32),
                   jax.ShapeDtypeStruct((1, 1), jnp.float32)),
        interpret=interpret,
    )(x2d, gate_w.T, gate_b.reshape(1, E))
    return slots, gates, plen, aux[0, 0]


def _sc_mesh():
    return plsc.VectorSubcoreMesh(core_axis_name="c", subcore_axis_name="s")


def _sc_wid():
    return lax.axis_index("s") * 2 + lax.axis_index("c")


def _scatter_body(x_hbm, slots_hbm, xs_hbm, idx_v, rows_v, sem):
    w = _sc_wid()
    pltpu.sync_copy(slots_hbm.at[w], idx_v)
    pltpu.sync_copy(x_hbm.at[pl.ds(w * TPW, TPW)], rows_v)
    pltpu.async_copy(rows_v, xs_hbm.at[idx_v.at[0]], sem).wait()
    pltpu.async_copy(rows_v, xs_hbm.at[idx_v.at[1]], sem).wait()


def _sc_scatter(x2d, slots_w):
    return pl.kernel(
        _scatter_body,
        out_type=jax.ShapeDtypeStruct((P_ROWS, D_MODEL), jnp.float32),
        mesh=_sc_mesh(),
        scratch_types=[
            pltpu.VMEM((TOP_K, TPW), jnp.int32),
            pltpu.VMEM((TPW, D_MODEL), jnp.float32),
            pltpu.SemaphoreType.DMA,
        ],
    )(x2d, slots_w)


def _gather_body(ys_hbm, slots_hbm, y1_hbm, y2_hbm, idx_v, rows_v, sem):
    w = _sc_wid()
    pltpu.sync_copy(slots_hbm.at[w], idx_v)
    pltpu.async_copy(ys_hbm.at[idx_v.at[0]], rows_v, sem).wait()
    pltpu.sync_copy(rows_v, y1_hbm.at[pl.ds(w * TPW, TPW)])
    pltpu.async_copy(ys_hbm.at[idx_v.at[1]], rows_v, sem).wait()
    pltpu.sync_copy(rows_v, y2_hbm.at[pl.ds(w * TPW, TPW)])


def _sc_gather(ys, slots_w):
    return pl.kernel(
        _gather_body,
        out_type=(jax.ShapeDtypeStruct((T_TOKENS, D_MODEL), jnp.float32),
                  jax.ShapeDtypeStruct((T_TOKENS, D_MODEL), jnp.float32)),
        mesh=_sc_mesh(),
        scratch_types=[
            pltpu.VMEM((TOP_K, TPW), jnp.int32),
            pltpu.VMEM((TPW, D_MODEL), jnp.float32),
            pltpu.SemaphoreType.DMA,
        ],
    )(ys, slots_w)


def _ffn_body(be_ref, nb_ref, xs_ref, w1_ref, w2_ref, out_ref):
    b = pl.program_id(0)

    @pl.when(b < nb_ref[0])
    def _():
        xb = xs_ref[...].astype(jnp.bfloat16)
        h = jax.nn.gelu(jnp.dot(xb, w1_ref[0],
                                preferred_element_type=jnp.float32))
        out_ref[...] = jnp.dot(h.astype(jnp.bfloat16), w2_ref[0],
                               preferred_element_type=jnp.float32)


def _ffn_grouped(xs, w1, w2, block_expert, nb, interpret=False):
    D, H = D_MODEL, D_HID

    def xs_map(b, be, nbr):
        return (jnp.minimum(b, nbr[0] - 1), 0)

    def w1_map(b, be, nbr):
        return (be[jnp.minimum(b, nbr[0] - 1)], 0, 0)

    def w2_map(b, be, nbr):
        return (be[jnp.minimum(b, nbr[0] - 1)], 0, 0)

    grid_spec = pltpu.PrefetchScalarGridSpec(
        num_scalar_prefetch=2,
        grid=(NB,),
        in_specs=[
            pl.BlockSpec((BM, D), xs_map),
            pl.BlockSpec((1, D, H), w1_map),
            pl.BlockSpec((1, H, D), w2_map),
        ],
        out_specs=pl.BlockSpec((BM, D), xs_map),
    )
    return pl.pallas_call(
        _ffn_body,
        grid_spec=grid_spec,
        out_shape=jax.ShapeDtypeStruct((P_ROWS, D), jnp.float32),
        interpret=interpret,
    )(block_expert, nb, xs, w1, w2)


def _combine_body(y1_ref, y2_ref, g_ref, out_ref):
    g = g_ref[...]
    out_ref[...] = g[:, 0:1] * y1_ref[...] + g[:, 1:2] * y2_ref[...]


def _combine(y1, y2, gates, interpret=False):
    T, D = T_TOKENS, D_MODEL
    RB = 512
    return pl.pallas_call(
        _combine_body,
        grid=(T // RB,),
        in_specs=[
            pl.BlockSpec((RB, D), lambda i: (i, 0)),
            pl.BlockSpec((RB, D), lambda i: (i, 0)),
            pl.BlockSpec((RB, TOP_K), lambda i: (i, 0)),
        ],
        out_specs=pl.BlockSpec((RB, D), lambda i: (i, 0)),
        out_shape=jax.ShapeDtypeStruct((T, D), jnp.float32),
        interpret=interpret,
    )(y1, y2, gates)


def _block_meta(plen):
    pl_row = plen.reshape(NUM_EXPERTS)
    off = jnp.cumsum(pl_row) - pl_row
    nb = jnp.sum(pl_row) // BM
    starts = jnp.arange(NB, dtype=jnp.int32) * BM
    inside = (starts[:, None] >= off[None, :]) & (
        starts[:, None] < (off + pl_row)[None, :])
    block_expert = jnp.sum(
        inside.astype(jnp.int32) * jnp.arange(NUM_EXPERTS, dtype=jnp.int32)[None, :],
        axis=1)
    return block_expert.astype(jnp.int32), nb.reshape(1).astype(jnp.int32)


def _moe(x, gate_w, gate_b, w1, w2):
    B, S, D = x.shape
    x2d = x.reshape(B * S, D)
    slots, gates, plen, aux = _router(x2d, gate_w, gate_b)
    block_expert, nb = _block_meta(plen)
    slots_w = slots.T.reshape(TOP_K, NW, TPW).transpose(1, 0, 2)
    xs = _sc_scatter(x2d, slots_w)
    ys = _ffn_grouped(xs, w1.astype(jnp.bfloat16), w2.astype(jnp.bfloat16),
                      block_expert, nb)
    y1, y2 = _sc_gather(ys, slots_w)
    out = _combine(y1, y2, gates)
    return out.reshape(B, S, D), aux


@jax.jit
def kernel(x, gate_w, gate_b, w1, w2):
    return _moe(x, gate_w, gate_b, w1, w2)


# speedup vs baseline: 1.3956x; 1.3956x over previous
import functools

import jax
import jax.numpy as jnp
from jax import lax
from jax.experimental import pallas as pl
from jax.experimental.pallas import tpu as pltpu
from jax.experimental.pallas import tpu_sc as plsc

NUM_EXPERTS = 8
TOP_K = 2
D_MODEL = 1024
D_HID = 2048
AUX_COEFF = 0.01

T_TOKENS = 2048
N_PAIRS = T_TOKENS * TOP_K
BM = 512
NB = (N_PAIRS + NUM_EXPERTS * BM) // BM
P_ROWS = NB * BM

NW = 32
TPW = T_TOKENS // NW


def _cumsum0(a):
    n = a.shape[0]
    sh = 1
    while sh < n:
        a = a + jnp.concatenate(
            [jnp.zeros((sh, a.shape[1]), a.dtype), a[:-sh]], axis=0)
        sh *= 2
    return a


def _router_body(x_ref, gwt_ref, gb_ref, slots1_ref, slots2_ref, gates_ref,
                 be_ref, nb_ref, aux_ref):
    T, E = T_TOKENS, NUM_EXPERTS
    logits = jnp.dot(x_ref[...], gwt_ref[...],
                     preferred_element_type=jnp.float32) + gb_ref[...]
    ids = jax.lax.broadcasted_iota(jnp.int32, (T, E), 1)
    m1 = jnp.max(logits, axis=1, keepdims=True)
    i1 = jnp.min(jnp.where(logits == m1, ids, E), axis=1, keepdims=True)
    neg = jnp.float32(-jnp.inf)
    logits_m = jnp.where(ids == i1, neg, logits)
    m2 = jnp.max(logits_m, axis=1, keepdims=True)
    i2 = jnp.min(jnp.where(logits_m == m2, ids, E), axis=1, keepdims=True)
    e21 = jnp.exp(m2 - m1)
    g1 = 1.0 / (1.0 + e21)
    g2 = e21 / (1.0 + e21)
    gates_ref[...] = jnp.concatenate([g1, g2], axis=1)
    lse = m1 + jnp.log(jnp.sum(jnp.exp(logits - m1), axis=1, keepdims=True))
    aux = (AUX_COEFF / E) * (-jnp.log(jnp.float32(E))
                             - jnp.mean(logits) + jnp.mean(lse))
    aux_ref[...] = jnp.reshape(aux, (1, 1))
    oh1 = (ids == i1).astype(jnp.int32)
    oh2 = (ids == i2).astype(jnp.int32)
    csum = _cumsum0(jnp.concatenate([oh1, oh2], axis=0))
    rank1 = jnp.sum(csum[:T] * oh1, axis=1, keepdims=True) - 1
    rank2 = jnp.sum(csum[T:] * oh2, axis=1, keepdims=True) - 1
    counts = csum[2 * T - 1:2 * T, :]
    plen = ((counts + (BM - 1)) // BM) * BM
    ec = jax.lax.broadcasted_iota(jnp.int32, (E, E), 1)
    er = jax.lax.broadcasted_iota(jnp.int32, (E, E), 0)
    off = jnp.sum(jnp.where(ec < er, jnp.broadcast_to(plen, (E, E)), 0),
                  axis=1).reshape(1, E)
    off_b = jnp.broadcast_to(off, (T, E))
    slots1_ref[...] = jnp.sum(oh1 * off_b, axis=1, keepdims=True) + rank1
    slots2_ref[...] = jnp.sum(oh2 * off_b, axis=1, keepdims=True) + rank2
    starts = jax.lax.broadcasted_iota(jnp.int32, (NB, E), 0) * BM
    off_nb = jnp.broadcast_to(off, (NB, E))
    plen_nb = jnp.broadcast_to(plen, (NB, E))
    e_nb = jax.lax.broadcasted_iota(jnp.int32, (NB, E), 1)
    inside = jnp.logical_and(starts >= off_nb, starts < off_nb + plen_nb)
    be_ref[...] = jnp.sum(jnp.where(inside, e_nb, 0), axis=1, keepdims=True)
    nb_ref[...] = jnp.sum(plen, axis=1, keepdims=True) // BM


def _router(x2d, gate_w, gate_b, interpret=False):
    T, E = T_TOKENS, NUM_EXPERTS
    return pl.pallas_call(
        _router_body,
        out_shape=(jax.ShapeDtypeStruct((T, 1), jnp.int32),
                   jax.ShapeDtypeStruct((T, 1), jnp.int32),
                   jax.ShapeDtypeStruct((T, TOP_K), jnp.float32),
                   jax.ShapeDtypeStruct((NB, 1), jnp.int32),
                   jax.ShapeDtypeStruct((1, 1), jnp.int32),
                   jax.ShapeDtypeStruct((1, 1), jnp.float32)),
        interpret=interpret,
    )(x2d, gate_w.T, gate_b.reshape(1, E))


def _sc_mesh():
    return plsc.VectorSubcoreMesh(core_axis_name="c", subcore_axis_name="s")


def _sc_wid():
    return lax.axis_index("s") * 2 + lax.axis_index("c")


def _scatter_body(x_hbm, s1_hbm, s2_hbm, xs_hbm, idx1_v, idx2_v, rows_v,
                  sem1, sem2):
    w = _sc_wid()
    pltpu.sync_copy(s1_hbm.at[w], idx1_v)
    pltpu.sync_copy(s2_hbm.at[w], idx2_v)
    pltpu.sync_copy(x_hbm.at[pl.ds(w * TPW, TPW)], rows_v)
    c1 = pltpu.async_copy(rows_v, xs_hbm.at[idx1_v], sem1)
    c2 = pltpu.async_copy(rows_v, xs_hbm.at[idx2_v], sem2)
    c1.wait()
    c2.wait()


def _sc_scatter(x2d, s1_w, s2_w):
    return pl.kernel(
        _scatter_body,
        out_type=jax.ShapeDtypeStruct((P_ROWS, D_MODEL), jnp.float32),
        mesh=_sc_mesh(),
        scratch_types=[
            pltpu.VMEM((TPW,), jnp.int32),
            pltpu.VMEM((TPW,), jnp.int32),
            pltpu.VMEM((TPW, D_MODEL), jnp.float32),
            pltpu.SemaphoreType.DMA,
            pltpu.SemaphoreType.DMA,
        ],
    )(x2d, s1_w, s2_w)


HPW = TPW // 2


def _gather_body(ys_hbm, s1_hbm, s2_hbm, y1_hbm, y2_hbm, idx1_v, idx2_v,
                 rows1_v, rows2_v, sem1, sem2):
    w = _sc_wid()
    pltpu.sync_copy(s1_hbm.at[w], idx1_v)
    pltpu.sync_copy(s2_hbm.at[w], idx2_v)
    c1 = pltpu.async_copy(ys_hbm.at[idx1_v], rows1_v, sem1)
    c2 = pltpu.async_copy(ys_hbm.at[idx2_v.at[pl.ds(0, HPW)]], rows2_v, sem2)
    c1.wait()
    pltpu.sync_copy(rows1_v, y1_hbm.at[pl.ds(w * TPW, TPW)])
    c2.wait()
    pltpu.sync_copy(rows2_v, y2_hbm.at[pl.ds(w * TPW, HPW)])
    c3 = pltpu.async_copy(ys_hbm.at[idx2_v.at[pl.ds(HPW, HPW)]], rows2_v,
                          sem2)
    c3.wait()
    pltpu.sync_copy(rows2_v, y2_hbm.at[pl.ds(w * TPW + HPW, HPW)])


def _sc_gather(ys, s1_w, s2_w):
    return pl.kernel(
        _gather_body,
        out_type=(jax.ShapeDtypeStruct((T_TOKENS, D_MODEL), jnp.float32),
                  jax.ShapeDtypeStruct((T_TOKENS, D_MODEL), jnp.float32)),
        mesh=_sc_mesh(),
        scratch_types=[
            pltpu.VMEM((TPW,), jnp.int32),
            pltpu.VMEM((TPW,), jnp.int32),
            pltpu.VMEM((TPW, D_MODEL), jnp.float32),
            pltpu.VMEM((HPW, D_MODEL), jnp.float32),
            pltpu.SemaphoreType.DMA,
            pltpu.SemaphoreType.DMA,
        ],
    )(ys, s1_w, s2_w)


def _ffn_body(be_ref, nb_ref, xs_ref, w1_ref, w2_ref, out_ref):
    b = pl.program_id(0)

    @pl.when(b < nb_ref[0])
    def _():
        h = jax.nn.gelu(jnp.dot(xs_ref[...], w1_ref[0],
                                preferred_element_type=jnp.float32))
        out_ref[...] = jnp.dot(h, w2_ref[0],
                               preferred_element_type=jnp.float32)


def _ffn_grouped(xs, w1, w2, block_expert, nb, interpret=False):
    D, H = D_MODEL, D_HID

    def xs_map(b, be, nbr):
        return (jnp.minimum(b, nbr[0] - 1), 0)

    def w1_map(b, be, nbr):
        return (be[jnp.minimum(b, nbr[0] - 1)], 0, 0)

    def w2_map(b, be, nbr):
        return (be[jnp.minimum(b, nbr[0] - 1)], 0, 0)

    grid_spec = pltpu.PrefetchScalarGridSpec(
        num_scalar_prefetch=2,
        grid=(NB,),
        in_specs=[
            pl.BlockSpec((BM, D), xs_map),
            pl.BlockSpec((1, D, H), w1_map),
            pl.BlockSpec((1, H, D), w2_map),
        ],
        out_specs=pl.BlockSpec((BM, D), xs_map),
    )
    return pl.pallas_call(
        _ffn_body,
        grid_spec=grid_spec,
        out_shape=jax.ShapeDtypeStruct((P_ROWS, D), jnp.float32),
        interpret=interpret,
    )(block_expert, nb, xs, w1, w2)


def _combine_body(y1_ref, y2_ref, g_ref, out_ref):
    g = g_ref[...]
    out_ref[...] = g[:, 0:1] * y1_ref[...] + g[:, 1:2] * y2_ref[...]


def _combine(y1, y2, gates, interpret=False):
    T, D = T_TOKENS, D_MODEL
    RB = 512
    return pl.pallas_call(
        _combine_body,
        grid=(T // RB,),
        in_specs=[
            pl.BlockSpec((RB, D), lambda i: (i, 0)),
            pl.BlockSpec((RB, D), lambda i: (i, 0)),
            pl.BlockSpec((RB, TOP_K), lambda i: (i, 0)),
        ],
        out_specs=pl.BlockSpec((RB, D), lambda i: (i, 0)),
        out_shape=jax.ShapeDtypeStruct((T, D), jnp.float32),
        interpret=interpret,
    )(y1, y2, gates)


def _moe(x, gate_w, gate_b, w1, w2):
    B, S, D = x.shape
    x2d = x.reshape(B * S, D)
    slots1, slots2, gates, be, nbv, aux = _router(x2d, gate_w, gate_b)
    block_expert = be.reshape(NB)
    nb = nbv.reshape(1)
    s1_w = slots1.reshape(NW, TPW)
    s2_w = slots2.reshape(NW, TPW)
    xs = _sc_scatter(x2d, s1_w, s2_w)
    ys = _ffn_grouped(xs, w1, w2, block_expert, nb)
    y1, y2 = _sc_gather(ys, s1_w, s2_w)
    out = _combine(y1, y2, gates)
    return out.reshape(B, S, D), aux[0, 0]


@jax.jit
def kernel(x, gate_w, gate_b, w1, w2):
    return _moe(x, gate_w, gate_b, w1, w2)


# speedup vs baseline: 1.5252x; 1.0928x over previous
import functools

import jax
import jax.numpy as jnp
from jax import lax
from jax.experimental import pallas as pl
from jax.experimental.pallas import tpu as pltpu
from jax.experimental.pallas import tpu_sc as plsc

NUM_EXPERTS = 8
TOP_K = 2
D_MODEL = 1024
D_HID = 2048
AUX_COEFF = 0.01

T_TOKENS = 2048
N_PAIRS = T_TOKENS * TOP_K
BM = 576
NB = -(-(N_PAIRS + NUM_EXPERTS * (BM - 1)) // BM)
P_ROWS = NB * BM

NW = 32
TPW = T_TOKENS // NW


def _cumsum0(a):
    n = a.shape[0]
    sh = 1
    while sh < n:
        a = a + jnp.concatenate(
            [jnp.zeros((sh, a.shape[1]), a.dtype), a[:-sh]], axis=0)
        sh *= 2
    return a


def _router_body(x_ref, gwt_ref, gb_ref, slots1_ref, slots2_ref, gates_ref,
                 be_ref, nb_ref, aux_ref):
    T, E = T_TOKENS, NUM_EXPERTS
    logits = jnp.dot(x_ref[...], gwt_ref[...],
                     preferred_element_type=jnp.float32) + gb_ref[...]
    ids = jax.lax.broadcasted_iota(jnp.int32, (T, E), 1)
    m1 = jnp.max(logits, axis=1, keepdims=True)
    i1 = jnp.min(jnp.where(logits == m1, ids, E), axis=1, keepdims=True)
    neg = jnp.float32(-jnp.inf)
    logits_m = jnp.where(ids == i1, neg, logits)
    m2 = jnp.max(logits_m, axis=1, keepdims=True)
    i2 = jnp.min(jnp.where(logits_m == m2, ids, E), axis=1, keepdims=True)
    e21 = jnp.exp(m2 - m1)
    g1 = 1.0 / (1.0 + e21)
    g2 = e21 / (1.0 + e21)
    gates_ref[...] = jnp.concatenate([g1, g2], axis=1)
    lse = m1 + jnp.log(jnp.sum(jnp.exp(logits - m1), axis=1, keepdims=True))
    aux = (AUX_COEFF / E) * (-jnp.log(jnp.float32(E))
                             - jnp.mean(logits) + jnp.mean(lse))
    aux_ref[...] = jnp.reshape(aux, (1, 1))
    oh1 = (ids == i1).astype(jnp.int32)
    oh2 = (ids == i2).astype(jnp.int32)
    csum = _cumsum0(jnp.concatenate([oh1, oh2], axis=0))
    rank1 = jnp.sum(csum[:T] * oh1, axis=1, keepdims=True) - 1
    rank2 = jnp.sum(csum[T:] * oh2, axis=1, keepdims=True) - 1
    counts = csum[2 * T - 1:2 * T, :]
    plen = ((counts + (BM - 1)) // BM) * BM
    ec = jax.lax.broadcasted_iota(jnp.int32, (E, E), 1)
    er = jax.lax.broadcasted_iota(jnp.int32, (E, E), 0)
    off = jnp.sum(jnp.where(ec < er, jnp.broadcast_to(plen, (E, E)), 0),
                  axis=1).reshape(1, E)
    off_b = jnp.broadcast_to(off, (T, E))
    slots1_ref[...] = jnp.sum(oh1 * off_b, axis=1, keepdims=True) + rank1
    slots2_ref[...] = jnp.sum(oh2 * off_b, axis=1, keepdims=True) + rank2
    starts = jax.lax.broadcasted_iota(jnp.int32, (NB, E), 0) * BM
    off_nb = jnp.broadcast_to(off, (NB, E))
    plen_nb = jnp.broadcast_to(plen, (NB, E))
    e_nb = jax.lax.broadcasted_iota(jnp.int32, (NB, E), 1)
    inside = jnp.logical_and(starts >= off_nb, starts < off_nb + plen_nb)
    be_ref[...] = jnp.sum(jnp.where(inside, e_nb, 0), axis=1, keepdims=True)
    nb_ref[...] = jnp.sum(plen, axis=1, keepdims=True) // BM


def _router(x2d, gate_w, gate_b, interpret=False):
    T, E = T_TOKENS, NUM_EXPERTS
    return pl.pallas_call(
        _router_body,
        out_shape=(jax.ShapeDtypeStruct((T, 1), jnp.int32),
                   jax.ShapeDtypeStruct((T, 1), jnp.int32),
                   jax.ShapeDtypeStruct((T, TOP_K), jnp.float32),
                   jax.ShapeDtypeStruct((NB, 1), jnp.int32),
                   jax.ShapeDtypeStruct((1, 1), jnp.int32),
                   jax.ShapeDtypeStruct((1, 1), jnp.float32)),
        interpret=interpret,
    )(x2d, gate_w.T, gate_b.reshape(1, E))


def _sc_mesh():
    return plsc.VectorSubcoreMesh(core_axis_name="c", subcore_axis_name="s")


def _sc_wid():
    return lax.axis_index("s") * 2 + lax.axis_index("c")


def _scatter_body(x_hbm, s1_hbm, s2_hbm, xs_hbm, idx1_v, idx2_v, rows_v,
                  sem1, sem2):
    w = _sc_wid()
    pltpu.sync_copy(s1_hbm.at[w], idx1_v)
    pltpu.sync_copy(s2_hbm.at[w], idx2_v)
    pltpu.sync_copy(x_hbm.at[pl.ds(w * TPW, TPW)], rows_v)
    c1 = pltpu.async_copy(rows_v, xs_hbm.at[idx1_v], sem1)
    c2 = pltpu.async_copy(rows_v, xs_hbm.at[idx2_v], sem2)
    c1.wait()
    c2.wait()


def _sc_scatter(x2d, s1_w, s2_w):
    return pl.kernel(
        _scatter_body,
        out_type=jax.ShapeDtypeStruct((P_ROWS, D_MODEL), jnp.float32),
        mesh=_sc_mesh(),
        scratch_types=[
            pltpu.VMEM((TPW,), jnp.int32),
            pltpu.VMEM((TPW,), jnp.int32),
            pltpu.VMEM((TPW, D_MODEL), jnp.float32),
            pltpu.SemaphoreType.DMA,
            pltpu.SemaphoreType.DMA,
        ],
    )(x2d, s1_w, s2_w)


HPW = TPW // 2


def _gather_body(ys_hbm, s1_hbm, s2_hbm, y1_hbm, y2_hbm, idx1_v, idx2_v,
                 rows1_v, rows2_v, sem1, sem2):
    w = _sc_wid()
    pltpu.sync_copy(s1_hbm.at[w], idx1_v)
    pltpu.sync_copy(s2_hbm.at[w], idx2_v)
    c1 = pltpu.async_copy(ys_hbm.at[idx1_v], rows1_v, sem1)
    c2 = pltpu.async_copy(ys_hbm.at[idx2_v.at[pl.ds(0, HPW)]], rows2_v, sem2)
    c1.wait()
    pltpu.sync_copy(rows1_v, y1_hbm.at[pl.ds(w * TPW, TPW)])
    c2.wait()
    pltpu.sync_copy(rows2_v, y2_hbm.at[pl.ds(w * TPW, HPW)])
    c3 = pltpu.async_copy(ys_hbm.at[idx2_v.at[pl.ds(HPW, HPW)]], rows2_v,
                          sem2)
    c3.wait()
    pltpu.sync_copy(rows2_v, y2_hbm.at[pl.ds(w * TPW + HPW, HPW)])


def _sc_gather(ys, s1_w, s2_w):
    return pl.kernel(
        _gather_body,
        out_type=(jax.ShapeDtypeStruct((T_TOKENS, D_MODEL), jnp.float32),
                  jax.ShapeDtypeStruct((T_TOKENS, D_MODEL), jnp.float32)),
        mesh=_sc_mesh(),
        scratch_types=[
            pltpu.VMEM((TPW,), jnp.int32),
            pltpu.VMEM((TPW,), jnp.int32),
            pltpu.VMEM((TPW, D_MODEL), jnp.float32),
            pltpu.VMEM((HPW, D_MODEL), jnp.float32),
            pltpu.SemaphoreType.DMA,
            pltpu.SemaphoreType.DMA,
        ],
    )(ys, s1_w, s2_w)


def _ffn_body(be_ref, nb_ref, xs_ref, w1_ref, w2_ref, out_ref):
    b = pl.program_id(0)

    @pl.when(b < nb_ref[0])
    def _():
        h = jax.nn.gelu(jnp.dot(xs_ref[...], w1_ref[0],
                                preferred_element_type=jnp.float32))
        out_ref[...] = jnp.dot(h, w2_ref[0],
                               preferred_element_type=jnp.float32)


def _ffn_grouped(xs, w1, w2, block_expert, nb, interpret=False):
    D, H = D_MODEL, D_HID

    def xs_map(b, be, nbr):
        return (jnp.minimum(b, nbr[0] - 1), 0)

    def w1_map(b, be, nbr):
        return (be[jnp.minimum(b, nbr[0] - 1)], 0, 0)

    def w2_map(b, be, nbr):
        return (be[jnp.minimum(b, nbr[0] - 1)], 0, 0)

    grid_spec = pltpu.PrefetchScalarGridSpec(
        num_scalar_prefetch=2,
        grid=(NB,),
        in_specs=[
            pl.BlockSpec((BM, D), xs_map),
            pl.BlockSpec((1, D, H), w1_map),
            pl.BlockSpec((1, H, D), w2_map),
        ],
        out_specs=pl.BlockSpec((BM, D), xs_map),
    )
    return pl.pallas_call(
        _ffn_body,
        grid_spec=grid_spec,
        out_shape=jax.ShapeDtypeStruct((P_ROWS, D), jnp.float32),
        interpret=interpret,
    )(block_expert, nb, xs, w1, w2)


def _combine_body(y1_ref, y2_ref, g_ref, out_ref):
    g = g_ref[...]
    out_ref[...] = g[:, 0:1] * y1_ref[...] + g[:, 1:2] * y2_ref[...]


def _combine(y1, y2, gates, interpret=False):
    T, D = T_TOKENS, D_MODEL
    RB = 512
    return pl.pallas_call(
        _combine_body,
        grid=(T // RB,),
        in_specs=[
            pl.BlockSpec((RB, D), lambda i: (i, 0)),
            pl.BlockSpec((RB, D), lambda i: (i, 0)),
            pl.BlockSpec((RB, TOP_K), lambda i: (i, 0)),
        ],
        out_specs=pl.BlockSpec((RB, D), lambda i: (i, 0)),
        out_shape=jax.ShapeDtypeStruct((T, D), jnp.float32),
        interpret=interpret,
    )(y1, y2, gates)


def _moe(x, gate_w, gate_b, w1, w2):
    B, S, D = x.shape
    x2d = x.reshape(B * S, D)
    slots1, slots2, gates, be, nbv, aux = _router(x2d, gate_w, gate_b)
    block_expert = be.reshape(NB)
    nb = nbv.reshape(1)
    s1_w = slots1.reshape(NW, TPW)
    s2_w = slots2.reshape(NW, TPW)
    xs = _sc_scatter(x2d, s1_w, s2_w)
    ys = _ffn_grouped(xs, w1, w2, block_expert, nb)
    y1, y2 = _sc_gather(ys, s1_w, s2_w)
    out = _combine(y1, y2, gates)
    return out.reshape(B, S, D), aux[0, 0]


@jax.jit
def kernel(x, gate_w, gate_b, w1, w2):
    return _moe(x, gate_w, gate_b, w1, w2)


# speedup vs baseline: 1.5270x; 1.0012x over previous
import functools

import jax
import jax.numpy as jnp
from jax import lax
from jax.experimental import pallas as pl
from jax.experimental.pallas import tpu as pltpu
from jax.experimental.pallas import tpu_sc as plsc

NUM_EXPERTS = 8
TOP_K = 2
D_MODEL = 1024
D_HID = 2048
AUX_COEFF = 0.01

T_TOKENS = 2048
N_PAIRS = T_TOKENS * TOP_K
BM = 576
NB = -(-(N_PAIRS + NUM_EXPERTS * (BM - 1)) // BM)
P_ROWS = NB * BM

NW = 32
TPW = T_TOKENS // NW


def _cumsum0(a):
    n = a.shape[0]
    sh = 1
    while sh < n:
        a = a + jnp.concatenate(
            [jnp.zeros((sh, a.shape[1]), a.dtype), a[:-sh]], axis=0)
        sh *= 2
    return a


def _router_body(x_ref, gwt_ref, gb_ref, slots1_ref, slots2_ref, gates_ref,
                 be_ref, nb_ref, aux_ref):
    T, E = T_TOKENS, NUM_EXPERTS
    logits = jnp.dot(x_ref[...], gwt_ref[...],
                     preferred_element_type=jnp.float32) + gb_ref[...]
    ids = jax.lax.broadcasted_iota(jnp.int32, (T, E), 1)
    m1 = jnp.max(logits, axis=1, keepdims=True)
    i1 = jnp.min(jnp.where(logits == m1, ids, E), axis=1, keepdims=True)
    neg = jnp.float32(-jnp.inf)
    logits_m = jnp.where(ids == i1, neg, logits)
    m2 = jnp.max(logits_m, axis=1, keepdims=True)
    i2 = jnp.min(jnp.where(logits_m == m2, ids, E), axis=1, keepdims=True)
    e21 = jnp.exp(m2 - m1)
    g1 = 1.0 / (1.0 + e21)
    g2 = e21 / (1.0 + e21)
    gates_ref[...] = jnp.concatenate([g1, g2], axis=1)
    lse = m1 + jnp.log(jnp.sum(jnp.exp(logits - m1), axis=1, keepdims=True))
    aux = (AUX_COEFF / E) * (-jnp.log(jnp.float32(E))
                             - jnp.mean(logits) + jnp.mean(lse))
    aux_ref[...] = jnp.reshape(aux, (1, 1))
    oh1 = (ids == i1).astype(jnp.int32)
    oh2 = (ids == i2).astype(jnp.int32)
    csum = _cumsum0(jnp.concatenate([oh1, oh2], axis=0))
    rank1 = jnp.sum(csum[:T] * oh1, axis=1, keepdims=True) - 1
    rank2 = jnp.sum(csum[T:] * oh2, axis=1, keepdims=True) - 1
    counts = csum[2 * T - 1:2 * T, :]
    plen = ((counts + (BM - 1)) // BM) * BM
    ec = jax.lax.broadcasted_iota(jnp.int32, (E, E), 1)
    er = jax.lax.broadcasted_iota(jnp.int32, (E, E), 0)
    off = jnp.sum(jnp.where(ec < er, jnp.broadcast_to(plen, (E, E)), 0),
                  axis=1).reshape(1, E)
    off_b = jnp.broadcast_to(off, (T, E))
    slots1_ref[...] = jnp.sum(oh1 * off_b, axis=1, keepdims=True) + rank1
    slots2_ref[...] = jnp.sum(oh2 * off_b, axis=1, keepdims=True) + rank2
    starts = jax.lax.broadcasted_iota(jnp.int32, (NB, E), 0) * BM
    off_nb = jnp.broadcast_to(off, (NB, E))
    plen_nb = jnp.broadcast_to(plen, (NB, E))
    e_nb = jax.lax.broadcasted_iota(jnp.int32, (NB, E), 1)
    inside = jnp.logical_and(starts >= off_nb, starts < off_nb + plen_nb)
    be_ref[...] = jnp.sum(jnp.where(inside, e_nb, 0), axis=1, keepdims=True)
    nb_ref[...] = jnp.sum(plen, axis=1, keepdims=True) // BM


def _router(x2d, gate_w, gate_b, interpret=False):
    T, E = T_TOKENS, NUM_EXPERTS
    return pl.pallas_call(
        _router_body,
        out_shape=(jax.ShapeDtypeStruct((T, 1), jnp.int32),
                   jax.ShapeDtypeStruct((T, 1), jnp.int32),
                   jax.ShapeDtypeStruct((T, TOP_K), jnp.float32),
                   jax.ShapeDtypeStruct((NB, 1), jnp.int32),
                   jax.ShapeDtypeStruct((1, 1), jnp.int32),
                   jax.ShapeDtypeStruct((1, 1), jnp.float32)),
        interpret=interpret,
    )(x2d, gate_w.T, gate_b.reshape(1, E))


def _sc_mesh():
    return plsc.VectorSubcoreMesh(core_axis_name="c", subcore_axis_name="s")


def _sc_wid():
    return lax.axis_index("s") * 2 + lax.axis_index("c")


def _scatter_body(x_hbm, s1_hbm, s2_hbm, xs_hbm, idx1_v, idx2_v, rows_v,
                  sem1, sem2):
    w = _sc_wid()
    pltpu.sync_copy(s1_hbm.at[w], idx1_v)
    pltpu.sync_copy(s2_hbm.at[w], idx2_v)
    pltpu.sync_copy(x_hbm.at[pl.ds(w * TPW, TPW)], rows_v)
    c1 = pltpu.async_copy(rows_v, xs_hbm.at[idx1_v], sem1)
    c2 = pltpu.async_copy(rows_v, xs_hbm.at[idx2_v], sem2)
    c1.wait()
    c2.wait()


def _sc_scatter(x2d, s1_w, s2_w):
    return pl.kernel(
        _scatter_body,
        out_type=jax.ShapeDtypeStruct((P_ROWS, D_MODEL), jnp.float32),
        mesh=_sc_mesh(),
        scratch_types=[
            pltpu.VMEM((TPW,), jnp.int32),
            pltpu.VMEM((TPW,), jnp.int32),
            pltpu.VMEM((TPW, D_MODEL), jnp.float32),
            pltpu.SemaphoreType.DMA,
            pltpu.SemaphoreType.DMA,
        ],
    )(x2d, s1_w, s2_w)


HPW = TPW // 2


def _gather_body(ys_hbm, s1_hbm, s2_hbm, y1_hbm, y2_hbm, idx1_v, idx2_v,
                 rowsa_v, rowsb_v, rowsc_v, sema, semb, semc):
    w = _sc_wid()
    pltpu.sync_copy(s1_hbm.at[w], idx1_v)
    pltpu.sync_copy(s2_hbm.at[w], idx2_v)
    ca = pltpu.async_copy(ys_hbm.at[idx1_v.at[pl.ds(0, HPW)]], rowsa_v, sema)
    cb = pltpu.async_copy(ys_hbm.at[idx1_v.at[pl.ds(HPW, HPW)]], rowsb_v,
                          semb)
    cc = pltpu.async_copy(ys_hbm.at[idx2_v.at[pl.ds(0, HPW)]], rowsc_v, semc)
    ca.wait()
    pltpu.sync_copy(rowsa_v, y1_hbm.at[pl.ds(w * TPW, HPW)])
    cd = pltpu.async_copy(ys_hbm.at[idx2_v.at[pl.ds(HPW, HPW)]], rowsa_v,
                          sema)
    cb.wait()
    pltpu.sync_copy(rowsb_v, y1_hbm.at[pl.ds(w * TPW + HPW, HPW)])
    cc.wait()
    pltpu.sync_copy(rowsc_v, y2_hbm.at[pl.ds(w * TPW, HPW)])
    cd.wait()
    pltpu.sync_copy(rowsa_v, y2_hbm.at[pl.ds(w * TPW + HPW, HPW)])


def _sc_gather(ys, s1_w, s2_w):
    return pl.kernel(
        _gather_body,
        out_type=(jax.ShapeDtypeStruct((T_TOKENS, D_MODEL), jnp.float32),
                  jax.ShapeDtypeStruct((T_TOKENS, D_MODEL), jnp.float32)),
        mesh=_sc_mesh(),
        scratch_types=[
            pltpu.VMEM((TPW,), jnp.int32),
            pltpu.VMEM((TPW,), jnp.int32),
            pltpu.VMEM((HPW, D_MODEL), jnp.float32),
            pltpu.VMEM((HPW, D_MODEL), jnp.float32),
            pltpu.VMEM((HPW, D_MODEL), jnp.float32),
            pltpu.SemaphoreType.DMA,
            pltpu.SemaphoreType.DMA,
            pltpu.SemaphoreType.DMA,
        ],
    )(ys, s1_w, s2_w)


def _ffn_body(be_ref, nb_ref, xs_ref, w1_ref, w2_ref, out_ref):
    b = pl.program_id(0)

    @pl.when(b < nb_ref[0])
    def _():
        h = jax.nn.gelu(jnp.dot(xs_ref[...], w1_ref[0],
                                preferred_element_type=jnp.float32))
        out_ref[...] = jnp.dot(h, w2_ref[0],
                               preferred_element_type=jnp.float32)


def _ffn_grouped(xs, w1, w2, block_expert, nb, interpret=False):
    D, H = D_MODEL, D_HID

    def xs_map(b, be, nbr):
        return (jnp.minimum(b, nbr[0] - 1), 0)

    def w1_map(b, be, nbr):
        return (be[jnp.minimum(b, nbr[0] - 1)], 0, 0)

    def w2_map(b, be, nbr):
        return (be[jnp.minimum(b, nbr[0] - 1)], 0, 0)

    grid_spec = pltpu.PrefetchScalarGridSpec(
        num_scalar_prefetch=2,
        grid=(NB,),
        in_specs=[
            pl.BlockSpec((BM, D), xs_map),
            pl.BlockSpec((1, D, H), w1_map),
            pl.BlockSpec((1, H, D), w2_map),
        ],
        out_specs=pl.BlockSpec((BM, D), xs_map),
    )
    return pl.pallas_call(
        _ffn_body,
        grid_spec=grid_spec,
        out_shape=jax.ShapeDtypeStruct((P_ROWS, D), jnp.float32),
        interpret=interpret,
    )(block_expert, nb, xs, w1, w2)


def _combine_body(y1_ref, y2_ref, g_ref, out_ref):
    g = g_ref[...]
    out_ref[...] = g[:, 0:1] * y1_ref[...] + g[:, 1:2] * y2_ref[...]


def _combine(y1, y2, gates, interpret=False):
    T, D = T_TOKENS, D_MODEL
    RB = 512
    return pl.pallas_call(
        _combine_body,
        grid=(T // RB,),
        in_specs=[
            pl.BlockSpec((RB, D), lambda i: (i, 0)),
            pl.BlockSpec((RB, D), lambda i: (i, 0)),
            pl.BlockSpec((RB, TOP_K), lambda i: (i, 0)),
        ],
        out_specs=pl.BlockSpec((RB, D), lambda i: (i, 0)),
        out_shape=jax.ShapeDtypeStruct((T, D), jnp.float32),
        interpret=interpret,
    )(y1, y2, gates)


def _moe(x, gate_w, gate_b, w1, w2):
    B, S, D = x.shape
    x2d = x.reshape(B * S, D)
    slots1, slots2, gates, be, nbv, aux = _router(x2d, gate_w, gate_b)
    block_expert = be.reshape(NB)
    nb = nbv.reshape(1)
    s1_w = slots1.reshape(NW, TPW)
    s2_w = slots2.reshape(NW, TPW)
    xs = _sc_scatter(x2d, s1_w, s2_w)
    ys = _ffn_grouped(xs, w1, w2, block_expert, nb)
    y1, y2 = _sc_gather(ys, s1_w, s2_w)
    out = _combine(y1, y2, gates)
    return out.reshape(B, S, D), aux[0, 0]


@jax.jit
def kernel(x, gate_w, gate_b, w1, w2):
    return _moe(x, gate_w, gate_b, w1, w2)


# speedup vs baseline: 1.5325x; 1.0036x over previous
import jax
import jax.numpy as jnp
from jax import lax
from jax.experimental import pallas as pl
from jax.experimental.pallas import tpu as pltpu
from jax.experimental.pallas import tpu_sc as plsc

NUM_EXPERTS = 8
TOP_K = 2
D_MODEL = 1024
D_HID = 2048
AUX_COEFF = 0.01

T_TOKENS = 2048
N_PAIRS = T_TOKENS * TOP_K
BM = 576
NB = -(-(N_PAIRS + NUM_EXPERTS * (BM - 1)) // BM)
P_ROWS = NB * BM

NW = 32
TPW = T_TOKENS // NW


def _cumsum0(a):
    n = a.shape[0]
    sh = 1
    while sh < n:
        a = a + jnp.concatenate(
            [jnp.zeros((sh, a.shape[1]), a.dtype), a[:-sh]], axis=0)
        sh *= 2
    return a


def _router_body(x_ref, gwt_ref, gb_ref, slots1_ref, slots2_ref, gates_ref,
                 be_ref, nb_ref, aux_ref):
    T, E = T_TOKENS, NUM_EXPERTS
    logits = jnp.dot(x_ref[...], gwt_ref[...],
                     preferred_element_type=jnp.float32) + gb_ref[...]
    ids = jax.lax.broadcasted_iota(jnp.int32, (T, E), 1)
    m1 = jnp.max(logits, axis=1, keepdims=True)
    i1 = jnp.min(jnp.where(logits == m1, ids, E), axis=1, keepdims=True)
    neg = jnp.float32(-jnp.inf)
    logits_m = jnp.where(ids == i1, neg, logits)
    m2 = jnp.max(logits_m, axis=1, keepdims=True)
    i2 = jnp.min(jnp.where(logits_m == m2, ids, E), axis=1, keepdims=True)
    e21 = jnp.exp(m2 - m1)
    g1 = 1.0 / (1.0 + e21)
    g2 = e21 / (1.0 + e21)
    gates_ref[...] = jnp.concatenate([g1, g2], axis=1)
    lse = m1 + jnp.log(jnp.sum(jnp.exp(logits - m1), axis=1, keepdims=True))
    aux = (AUX_COEFF / E) * (-jnp.log(jnp.float32(E))
                             - jnp.mean(logits) + jnp.mean(lse))
    aux_ref[...] = jnp.reshape(aux, (1, 1))
    oh1 = (ids == i1).astype(jnp.int32)
    oh2 = (ids == i2).astype(jnp.int32)
    csum = _cumsum0(jnp.concatenate([oh1, oh2], axis=0))
    rank1 = jnp.sum(csum[:T] * oh1, axis=1, keepdims=True) - 1
    rank2 = jnp.sum(csum[T:] * oh2, axis=1, keepdims=True) - 1
    counts = csum[2 * T - 1:2 * T, :]
    plen = ((counts + (BM - 1)) // BM) * BM
    ec = jax.lax.broadcasted_iota(jnp.int32, (E, E), 1)
    er = jax.lax.broadcasted_iota(jnp.int32, (E, E), 0)
    off = jnp.sum(jnp.where(ec < er, jnp.broadcast_to(plen, (E, E)), 0),
                  axis=1).reshape(1, E)
    off_b = jnp.broadcast_to(off, (T, E))
    slots1_ref[...] = jnp.sum(oh1 * off_b, axis=1, keepdims=True) + rank1
    slots2_ref[...] = jnp.sum(oh2 * off_b, axis=1, keepdims=True) + rank2
    starts = jax.lax.broadcasted_iota(jnp.int32, (NB, E), 0) * BM
    off_nb = jnp.broadcast_to(off, (NB, E))
    plen_nb = jnp.broadcast_to(plen, (NB, E))
    e_nb = jax.lax.broadcasted_iota(jnp.int32, (NB, E), 1)
    inside = jnp.logical_and(starts >= off_nb, starts < off_nb + plen_nb)
    be_ref[...] = jnp.sum(jnp.where(inside, e_nb, 0), axis=1, keepdims=True)
    nb_ref[...] = jnp.sum(plen, axis=1, keepdims=True) // BM


def _router(x2d, gate_w, gate_b, interpret=False):
    T, E = T_TOKENS, NUM_EXPERTS
    return pl.pallas_call(
        _router_body,
        out_shape=(jax.ShapeDtypeStruct((T, 1), jnp.int32),
                   jax.ShapeDtypeStruct((T, 1), jnp.int32),
                   jax.ShapeDtypeStruct((T, TOP_K), jnp.float32),
                   jax.ShapeDtypeStruct((NB, 1), jnp.int32),
                   jax.ShapeDtypeStruct((1, 1), jnp.int32),
                   jax.ShapeDtypeStruct((1, 1), jnp.float32)),
        interpret=interpret,
    )(x2d, gate_w.T, gate_b.reshape(1, E))


def _sc_mesh():
    return plsc.VectorSubcoreMesh(core_axis_name="c", subcore_axis_name="s")


def _sc_wid():
    return lax.axis_index("s") * 2 + lax.axis_index("c")


def _scatter_body(x_hbm, s1_hbm, s2_hbm, xs_hbm, idx1_v, idx2_v, rows_v,
                  sem1, sem2):
    w = _sc_wid()
    pltpu.sync_copy(s1_hbm.at[w], idx1_v)
    pltpu.sync_copy(s2_hbm.at[w], idx2_v)
    pltpu.sync_copy(x_hbm.at[pl.ds(w * TPW, TPW)], rows_v)
    c1 = pltpu.async_copy(rows_v, xs_hbm.at[idx1_v], sem1)
    c2 = pltpu.async_copy(rows_v, xs_hbm.at[idx2_v], sem2)
    c1.wait()
    c2.wait()


def _sc_scatter(x2d, s1_w, s2_w):
    return pl.kernel(
        _scatter_body,
        out_type=jax.ShapeDtypeStruct((P_ROWS, D_MODEL), jnp.float32),
        mesh=_sc_mesh(),
        scratch_types=[
            pltpu.VMEM((TPW,), jnp.int32),
            pltpu.VMEM((TPW,), jnp.int32),
            pltpu.VMEM((TPW, D_MODEL), jnp.float32),
            pltpu.SemaphoreType.DMA,
            pltpu.SemaphoreType.DMA,
        ],
    )(x2d, s1_w, s2_w)


HPW = TPW // 2


def _gather_body(ys_hbm, s1_hbm, s2_hbm, y1_hbm, y2_hbm, idx1_v, idx2_v,
                 rowsa_v, rowsb_v, rowsc_v, sema, semb, semc):
    w = _sc_wid()
    pltpu.sync_copy(s1_hbm.at[w], idx1_v)
    pltpu.sync_copy(s2_hbm.at[w], idx2_v)
    ca = pltpu.async_copy(ys_hbm.at[idx1_v.at[pl.ds(0, HPW)]], rowsa_v, sema)
    cb = pltpu.async_copy(ys_hbm.at[idx1_v.at[pl.ds(HPW, HPW)]], rowsb_v,
                          semb)
    cc = pltpu.async_copy(ys_hbm.at[idx2_v.at[pl.ds(0, HPW)]], rowsc_v, semc)
    ca.wait()
    pltpu.sync_copy(rowsa_v, y1_hbm.at[pl.ds(w * TPW, HPW)])
    cd = pltpu.async_copy(ys_hbm.at[idx2_v.at[pl.ds(HPW, HPW)]], rowsa_v,
                          sema)
    cb.wait()
    pltpu.sync_copy(rowsb_v, y1_hbm.at[pl.ds(w * TPW + HPW, HPW)])
    cc.wait()
    pltpu.sync_copy(rowsc_v, y2_hbm.at[pl.ds(w * TPW, HPW)])
    cd.wait()
    pltpu.sync_copy(rowsa_v, y2_hbm.at[pl.ds(w * TPW + HPW, HPW)])


def _sc_gather(ys, s1_w, s2_w):
    return pl.kernel(
        _gather_body,
        out_type=(jax.ShapeDtypeStruct((T_TOKENS, D_MODEL), jnp.float32),
                  jax.ShapeDtypeStruct((T_TOKENS, D_MODEL), jnp.float32)),
        mesh=_sc_mesh(),
        scratch_types=[
            pltpu.VMEM((TPW,), jnp.int32),
            pltpu.VMEM((TPW,), jnp.int32),
            pltpu.VMEM((HPW, D_MODEL), jnp.float32),
            pltpu.VMEM((HPW, D_MODEL), jnp.float32),
            pltpu.VMEM((HPW, D_MODEL), jnp.float32),
            pltpu.SemaphoreType.DMA,
            pltpu.SemaphoreType.DMA,
            pltpu.SemaphoreType.DMA,
        ],
    )(ys, s1_w, s2_w)


def _ffn_body(be_ref, nb_ref, xs_ref, w1_ref, w2_ref, out_ref):
    b = pl.program_id(0)

    @pl.when(b < nb_ref[0])
    def _():
        h = jax.nn.gelu(jnp.dot(xs_ref[...], w1_ref[0],
                                preferred_element_type=jnp.float32))
        out_ref[...] = jnp.dot(h, w2_ref[0],
                               preferred_element_type=jnp.float32)


def _ffn_grouped(xs, w1, w2, block_expert, nb, interpret=False):
    D, H = D_MODEL, D_HID

    def xs_map(b, be, nbr):
        return (jnp.minimum(b, nbr[0] - 1), 0)

    def w1_map(b, be, nbr):
        return (be[jnp.minimum(b, nbr[0] - 1)], 0, 0)

    def w2_map(b, be, nbr):
        return (be[jnp.minimum(b, nbr[0] - 1)], 0, 0)

    grid_spec = pltpu.PrefetchScalarGridSpec(
        num_scalar_prefetch=2,
        grid=(NB,),
        in_specs=[
            pl.BlockSpec((BM, D), xs_map),
            pl.BlockSpec((1, D, H), w1_map),
            pl.BlockSpec((1, H, D), w2_map),
        ],
        out_specs=pl.BlockSpec((BM, D), xs_map),
    )
    return pl.pallas_call(
        _ffn_body,
        grid_spec=grid_spec,
        out_shape=jax.ShapeDtypeStruct((P_ROWS, D), jnp.float32),
        interpret=interpret,
    )(block_expert, nb, xs, w1, w2)


def _combine_body(y1_ref, y2_ref, g_ref, out_ref):
    g = g_ref[...]
    out_ref[...] = g[:, 0:1] * y1_ref[...] + g[:, 1:2] * y2_ref[...]


def _combine(y1, y2, gates, interpret=False):
    T, D = T_TOKENS, D_MODEL
    RB = 512
    return pl.pallas_call(
        _combine_body,
        grid=(T // RB,),
        in_specs=[
            pl.BlockSpec((RB, D), lambda i: (i, 0)),
            pl.BlockSpec((RB, D), lambda i: (i, 0)),
            pl.BlockSpec((RB, TOP_K), lambda i: (i, 0)),
        ],
        out_specs=pl.BlockSpec((RB, D), lambda i: (i, 0)),
        out_shape=jax.ShapeDtypeStruct((T, D), jnp.float32),
        interpret=interpret,
    )(y1, y2, gates)


def _moe(x, gate_w, gate_b, w1, w2):
    B, S, D = x.shape
    x2d = x.reshape(B * S, D)
    slots1, slots2, gates, be, nbv, aux = _router(x2d, gate_w, gate_b)
    block_expert = be.reshape(NB)
    nb = nbv.reshape(1)
    s1_w = slots1.reshape(NW, TPW)
    s2_w = slots2.reshape(NW, TPW)
    xs = _sc_scatter(x2d, s1_w, s2_w)
    ys = _ffn_grouped(xs, w1, w2, block_expert, nb)
    y1, y2 = _sc_gather(ys, s1_w, s2_w)
    out = _combine(y1, y2, gates)
    return out.reshape(B, S, D), aux[0, 0]


@jax.jit
def kernel(x, gate_w, gate_b, w1, w2):
    return _moe(x, gate_w, gate_b, w1, w2)
